# baseline (speedup 1.0000x reference)
import functools

import jax
import jax.numpy as jnp
from jax import lax
from jax.experimental import pallas as pl
from jax.experimental.pallas import tpu as pltpu
from jax.experimental.pallas import tpu_sc as plsc

_NUM_W = 1000000
_DIM = 64
_NUM_C = 1024
_B = 4096
_T_LEN = 20
_BODY_LEN = 200

_NC, _NS = 2, 16
_NW = _NC * _NS
_SPW = _B // _NW
_PAIRS = _SPW // 2
_TP = 32
_BP = 208
_TPP = 2 * _TP
_BPP = 2 * _BP


def _sc_body(w2, tidx_hbm, bidx_hbm, tsum_hbm, bsum_hbm,
             tidx_v, bidx_v, tg_v, bg_v, trows_v, brows_v,
             tout_v, bout_v, sem):
    wid = lax.axis_index("s") * _NC + lax.axis_index("c")
    sbase = wid * _SPW
    pltpu.sync_copy(tidx_hbm.at[pl.ds(wid * _SPW * _TP, _SPW * _TP)],
                    tidx_v)
    pltpu.sync_copy(bidx_hbm.at[pl.ds(wid * _SPW * _BP, _SPW * _BP)],
                    bidx_v)

    def pair_body(j, carry):
        toff = pl.multiple_of(j * _TPP, 8)
        boff = pl.multiple_of(j * _BPP, 8)
        for o in range(0, _TPP, 16):
            tg_v[pl.ds(o, 16)] = lax.shift_right_logical(
                tidx_v[pl.ds(toff + o, 16)], 1)
        for o in range(0, _BPP, 16):
            bg_v[pl.ds(o, 16)] = lax.shift_right_logical(
                bidx_v[pl.ds(boff + o, 16)], 1)
        cps = []
        for s in range(2):
            cps.append(pltpu.async_copy(
                w2.at[tg_v.at[pl.ds(s * _TP, _T_LEN)]],
                trows_v.at[pl.ds(s * _TP, _T_LEN)], sem))
            for (o, ln) in ((0, 80), (80, 80), (160, 40)):
                cps.append(pltpu.async_copy(
                    w2.at[bg_v.at[pl.ds(s * _BP + o, ln)]],
                    brows_v.at[pl.ds(s * _BP + o, ln)], sem))
        for cp in cps:
            cp.wait()
        for s in range(2):
            tacc = [jnp.zeros((16,), jnp.float32) for _ in range(4)]
            for blk, nk in ((0, 16), (1, 4)):
                raw16 = tidx_v[pl.ds(toff + s * _TP + blk * 16, 16)]
                half16 = (raw16 & 1).astype(jnp.float32)
                for k in range(nk):
                    r = s * _TP + blk * 16 + k
                    f = jnp.full((16,), half16[k], jnp.float32)
                    for d in range(4):
                        v0 = trows_v[r, pl.ds(d * 16, 16)]
                        v1 = trows_v[r, pl.ds(_DIM + d * 16, 16)]
                        tacc[d] = tacc[d] + (v0 + f * (v1 - v0))

            def red16(it, accs):
                a = list(accs)
                base = s * _BP + it * 16
                raw16 = bidx_v[pl.ds(boff + base, 16)]
                half16 = (raw16 & 1).astype(jnp.float32)
                for k in range(16):
                    f = jnp.full((16,), half16[k], jnp.float32)
                    for d in range(4):
                        v0 = brows_v[base + k, pl.ds(d * 16, 16)]
                        v1 = brows_v[base + k, pl.ds(_DIM + d * 16, 16)]
                        a[d] = a[d] + (v0 + f * (v1 - v0))
                return tuple(a)
            bacc = lax.fori_loop(
                0, 12, red16,
                tuple(jnp.zeros((16,), jnp.float32) for _ in range(4)))
            bacc = list(bacc)
            tbase = s * _BP + 192
            raw16 = bidx_v[pl.ds(boff + tbase, 16)]
            half16 = (raw16 & 1).astype(jnp.float32)
            for k in range(8):
                f = jnp.full((16,), half16[k], jnp.float32)
                for d in range(4):
                    v0 = brows_v[tbase + k, pl.ds(d * 16, 16)]
                    v1 = brows_v[tbase + k, pl.ds(_DIM + d * 16, 16)]
                    bacc[d] = bacc[d] + (v0 + f * (v1 - v0))
            row = (2 * j + s) % 16
            for d in range(4):
                tout_v[row, pl.ds(d * 16, 16)] = tacc[d]
                bout_v[row, pl.ds(d * 16, 16)] = bacc[d]
        @pl.when((j % 8) == 7)
        def _():
            g = pl.multiple_of(sbase + (j - 7) * 2, 8)
            pltpu.sync_copy(tout_v, tsum_hbm.at[pl.ds(g, 16)])
            pltpu.sync_copy(bout_v, bsum_hbm.at[pl.ds(g, 16)])
        return carry

    lax.fori_loop(0, _PAIRS, pair_body, 0)


def _sc_pool(W, title2, body2):
    mesh = plsc.VectorSubcoreMesh(core_axis_name="c", subcore_axis_name="s",
                                  num_cores=_NC, num_subcores=_NS)
    f = pl.kernel(
        _sc_body,
        out_type=(jax.ShapeDtypeStruct((_B, _DIM), jnp.float32),
                  jax.ShapeDtypeStruct((_B, _DIM), jnp.float32)),
        mesh=mesh,
        scratch_types=[
            pltpu.VMEM((_SPW * _TP,), jnp.int32),
            pltpu.VMEM((_SPW * _BP,), jnp.int32),
            pltpu.VMEM((_TPP,), jnp.int32),
            pltpu.VMEM((_BPP,), jnp.int32),
            pltpu.VMEM((_TPP, 2 * _DIM), jnp.float32),
            pltpu.VMEM((_BPP, 2 * _DIM), jnp.float32),
            pltpu.VMEM((16, _DIM), jnp.float32),
            pltpu.VMEM((16, _DIM), jnp.float32),
            pltpu.SemaphoreType.DMA,
        ],
    )
    return f(W, title2, body2)


_RCH = 128
_RBASE = 31248
_RIT = -(-(_RBASE + 64) // _RCH)


def _sc_repack_body(w_hbm, wc_hbm, vin0, vin1, vout0, vout1, isem, osem):
    wid = lax.axis_index("s") * _NC + lax.axis_index("c")
    base = wid * _RBASE
    limit = jnp.where(wid == _NW - 1, _NUM_W, base + _RBASE)
    vins = (vin0, vin1)
    vouts = (vout0, vout1)

    def start(j):
        return pl.multiple_of(jnp.minimum(base + j * _RCH, limit - _RCH), 8)

    def issue_in(j, b):
        pltpu.async_copy(w_hbm.at[pl.ds(start(j), _RCH)], vins[b], isem)

    issue_in(0, 0)

    def step2(ii, carry):
        for b in range(2):
            j = 2 * ii + b

            @pl.when(j < _RIT)
            def _():
                @pl.when(j + 1 < _RIT)
                def _():
                    issue_in(j + 1, 1 - b)
                pltpu.make_async_copy(
                    w_hbm.at[pl.ds(0, _RCH)], vins[b], isem).wait()
                @pl.when(j >= 2)
                def _():
                    pltpu.make_async_copy(
                        vouts[b], wc_hbm.at[pl.ds(0, _RCH // 2)],
                        osem).wait()
                vin = vins[b]
                vout = vouts[b]
                for g in range(_RCH // 2):
                    for d in range(4):
                        vout[g, pl.ds(d * 16, 16)] = vin[2 * g,
                                                         pl.ds(d * 16, 16)]
                        vout[g, pl.ds(_DIM + d * 16, 16)] = vin[
                            2 * g + 1, pl.ds(d * 16, 16)]
                so = pl.multiple_of(start(j) // 2, 8)
                pltpu.async_copy(vout, wc_hbm.at[pl.ds(so, _RCH // 2)],
                                 osem)
        return carry

    lax.fori_loop(0, (_RIT + 1) // 2, step2, 0)
    for b in range(2):
        pltpu.make_async_copy(
            vouts[b], wc_hbm.at[pl.ds(0, _RCH // 2)], osem).wait()


def _sc_repack(W):
    mesh = plsc.VectorSubcoreMesh(core_axis_name="c", subcore_axis_name="s",
                                  num_cores=_NC, num_subcores=_NS)
    f = pl.kernel(
        _sc_repack_body,
        out_type=jax.ShapeDtypeStruct((_NUM_W // 2, 2 * _DIM), jnp.float32),
        mesh=mesh,
        scratch_types=[
            pltpu.VMEM((_RCH, _DIM), jnp.float32),
            pltpu.VMEM((_RCH, _DIM), jnp.float32),
            pltpu.VMEM((_RCH // 2, 2 * _DIM), jnp.float32),
            pltpu.VMEM((_RCH // 2, 2 * _DIM), jnp.float32),
            pltpu.SemaphoreType.DMA,
            pltpu.SemaphoreType.DMA,
        ],
    )
    return f(W)


def _tc_body(nblk, tsum_ref, bsum_ref, tint_ref, bint_ref, c_ref, out_ref,
             acc_rep, acc_rec, acc_d):
    i = pl.program_id(0)

    @pl.when(i == 0)
    def _():
        acc_rep[...] = jnp.zeros_like(acc_rep)
        acc_rec[...] = jnp.zeros_like(acc_rec)
        acc_d[0] = 0.0

    tcnt = jnp.sum((tint_ref[...] > 0).astype(jnp.float32), axis=1,
                   keepdims=True)
    bcnt = jnp.sum((bint_ref[...] > 0).astype(jnp.float32), axis=1,
                   keepdims=True)
    que = 0.3 * tsum_ref[...] / tcnt + 0.7 * bsum_ref[...] / bcnt
    cmat = c_ref[...]
    score = lax.dot_general(que, cmat, (((1,), (1,)), ((), ())),
                            preferred_element_type=jnp.float32)
    m = jnp.max(score, axis=1, keepdims=True)
    e = jnp.exp(score - m)
    probs = e / jnp.sum(e, axis=1, keepdims=True)
    rec = lax.dot_general(probs, cmat, (((1,), (0,)), ((), ())),
                          preferred_element_type=jnp.float32)
    n_rep = jnp.sqrt(jnp.sum(que * que, axis=1, keepdims=True))
    n_rec = jnp.sqrt(jnp.sum(rec * rec, axis=1, keepdims=True))
    denom = jnp.maximum(n_rec * n_rep, 1e-8)
    diag = jnp.sum(rec * que, axis=1, keepdims=True) / denom
    rep_hat = que / jnp.maximum(n_rep, 1e-20)
    rec_hat = rec / jnp.maximum(n_rec, 1e-20)

    acc_rep[...] = acc_rep[...] + jnp.sum(rep_hat, axis=0, keepdims=True)
    acc_rec[...] = acc_rec[...] + jnp.sum(rec_hat, axis=0, keepdims=True)
    acc_d[0] = acc_d[0] + jnp.sum(diag)

    @pl.when(i == nblk - 1)
    def _():
        total = (jnp.float32(_B) * jnp.float32(_B)
                 + jnp.sum(acc_rep[...] * acc_rec[...])
                 - 2.0 * acc_d[0])
        out_ref[...] = jnp.full((1, 1), total, jnp.float32)


def _tc_loss(tsum, bsum, title_int, body_int, C):
    blk = 512
    nblk = _B // blk
    return pl.pallas_call(
        functools.partial(_tc_body, nblk),
        grid=(nblk,),
        in_specs=[
            pl.BlockSpec((blk, _DIM), lambda i: (i, 0)),
            pl.BlockSpec((blk, _DIM), lambda i: (i, 0)),
            pl.BlockSpec((blk, _T_LEN), lambda i: (i, 0)),
            pl.BlockSpec((blk, _BODY_LEN), lambda i: (i, 0)),
            pl.BlockSpec((_NUM_C, _DIM), lambda i: (0, 0)),
        ],
        out_specs=pl.BlockSpec((1, 1), lambda i: (0, 0)),
        out_shape=jax.ShapeDtypeStruct((1, 1), jnp.float32),
        scratch_shapes=[
            pltpu.VMEM((1, _DIM), jnp.float32),
            pltpu.VMEM((1, _DIM), jnp.float32),
            pltpu.SMEM((1,), jnp.float32),
        ],
        compiler_params=pltpu.CompilerParams(
            dimension_semantics=("arbitrary",)),
    )(tsum, bsum, title_int, body_int, C)


def kernel(title_int, body_int, user_int, W, C):
    title2 = jnp.pad(title_int.astype(jnp.int32),
                     ((0, 0), (0, _TP - _T_LEN))).reshape(_B * _TP)
    body2 = jnp.pad(body_int.astype(jnp.int32),
                    ((0, 0), (0, _BP - _BODY_LEN))).reshape(_B * _BP)
    w2 = _sc_repack(W)
    tsum, bsum = _sc_pool(w2, title2, body2)
    out = _tc_loss(tsum, bsum, title_int.astype(jnp.int32),
                   body_int.astype(jnp.int32), C)
    return out[0, 0]


# speedup vs baseline: 1.4158x; 1.4158x over previous
import functools

import jax
import jax.numpy as jnp
from jax import lax
from jax.experimental import pallas as pl
from jax.experimental.pallas import tpu as pltpu
from jax.experimental.pallas import tpu_sc as plsc

_NUM_W = 1000000
_DIM = 64
_NUM_C = 1024
_B = 4096
_T_LEN = 20
_BODY_LEN = 200

_NC, _NS = 2, 16
_NW = _NC * _NS
_SPW = _B // _NW
_PAIRS = _SPW // 2
_TPP = 2 * _T_LEN
_BPP = 2 * _BODY_LEN
_BCH = 80
_NBCH = _BPP // _BCH


def _sc_body(w_hbm, tidx_hbm, bidx_hbm, tsum_hbm, bsum_hbm,
             tidx_v, bidx_v, trows_v, brows_v, tout_v, bout_v, sem):
    wid = lax.axis_index("s") * _NC + lax.axis_index("c")
    sbase = wid * _SPW
    pltpu.sync_copy(tidx_hbm.at[pl.ds(wid * _SPW * _T_LEN, _SPW * _T_LEN)],
                    tidx_v)
    pltpu.sync_copy(bidx_hbm.at[pl.ds(wid * _SPW * _BODY_LEN,
                                      _SPW * _BODY_LEN)], bidx_v)

    def pair_body(j, carry):
        toff = pl.multiple_of(j * _TPP, 8)
        cps = [pltpu.async_copy(w_hbm.at[tidx_v.at[pl.ds(toff, _TPP)]],
                                trows_v, sem)]
        for c in range(_NBCH):
            boff = pl.multiple_of(j * _BPP + c * _BCH, 8)
            cps.append(pltpu.async_copy(
                w_hbm.at[bidx_v.at[pl.ds(boff, _BCH)]],
                brows_v.at[pl.ds(c * _BCH, _BCH)], sem))
        for cp in cps:
            cp.wait()
        for s in range(2):
            tacc = [jnp.zeros((16,), jnp.float32) for _ in range(4)]
            for r in range(_T_LEN):
                for d in range(4):
                    tacc[d] = tacc[d] + trows_v[s * _T_LEN + r,
                                                pl.ds(d * 16, 16)]
            def red8(it, accs):
                a = list(accs)
                for k in range(8):
                    r = s * _BODY_LEN + it * 8 + k
                    for d in range(4):
                        a[d] = a[d] + brows_v[r, pl.ds(d * 16, 16)]
                return tuple(a)
            bacc = lax.fori_loop(
                0, _BODY_LEN // 8, red8,
                tuple(jnp.zeros((16,), jnp.float32) for _ in range(4)))
            row = 2 * j + s
            for d in range(4):
                tout_v[row, pl.ds(d * 16, 16)] = tacc[d]
                bout_v[row, pl.ds(d * 16, 16)] = bacc[d]
        return carry

    lax.fori_loop(0, _PAIRS, pair_body, 0)
    pltpu.sync_copy(tout_v, tsum_hbm.at[pl.ds(sbase, _SPW)])
    pltpu.sync_copy(bout_v, bsum_hbm.at[pl.ds(sbase, _SPW)])


def _sc_pool(W, title2, body2):
    mesh = plsc.VectorSubcoreMesh(core_axis_name="c", subcore_axis_name="s",
                                  num_cores=_NC, num_subcores=_NS)
    f = pl.kernel(
        _sc_body,
        out_type=(jax.ShapeDtypeStruct((_B, _DIM), jnp.float32),
                  jax.ShapeDtypeStruct((_B, _DIM), jnp.float32)),
        mesh=mesh,
        scratch_types=[
            pltpu.VMEM((_SPW * _T_LEN,), jnp.int32),
            pltpu.VMEM((_SPW * _BODY_LEN,), jnp.int32),
            pltpu.VMEM((_TPP, _DIM), jnp.float32),
            pltpu.VMEM((_BPP, _DIM), jnp.float32),
            pltpu.VMEM((_SPW, _DIM), jnp.float32),
            pltpu.VMEM((_SPW, _DIM), jnp.float32),
            pltpu.SemaphoreType.DMA,
        ],
        compiler_params=pltpu.CompilerParams(use_tc_tiling_on_sc=False),
    )
    return f(W, title2, body2)


def _tc_body(nblk, tsum_ref, bsum_ref, tint_ref, bint_ref, c_ref, out_ref,
             acc_rep, acc_rec, acc_d):
    i = pl.program_id(0)

    @pl.when(i == 0)
    def _():
        acc_rep[...] = jnp.zeros_like(acc_rep)
        acc_rec[...] = jnp.zeros_like(acc_rec)
        acc_d[0] = 0.0

    tcnt = jnp.sum((tint_ref[...] > 0).astype(jnp.float32), axis=1,
                   keepdims=True)
    bcnt = jnp.sum((bint_ref[...] > 0).astype(jnp.float32), axis=1,
                   keepdims=True)
    que = 0.3 * tsum_ref[...] / tcnt + 0.7 * bsum_ref[...] / bcnt
    cmat = c_ref[...]
    score = lax.dot_general(que, cmat, (((1,), (1,)), ((), ())),
                            preferred_element_type=jnp.float32)
    m = jnp.max(score, axis=1, keepdims=True)
    e = jnp.exp(score - m)
    probs = e / jnp.sum(e, axis=1, keepdims=True)
    rec = lax.dot_general(probs, cmat, (((1,), (0,)), ((), ())),
                          preferred_element_type=jnp.float32)
    n_rep = jnp.sqrt(jnp.sum(que * que, axis=1, keepdims=True))
    n_rec = jnp.sqrt(jnp.sum(rec * rec, axis=1, keepdims=True))
    denom = jnp.maximum(n_rec * n_rep, 1e-8)
    diag = jnp.sum(rec * que, axis=1, keepdims=True) / denom
    rep_hat = que / jnp.maximum(n_rep, 1e-20)
    rec_hat = rec / jnp.maximum(n_rec, 1e-20)

    acc_rep[...] = acc_rep[...] + jnp.sum(rep_hat, axis=0, keepdims=True)
    acc_rec[...] = acc_rec[...] + jnp.sum(rec_hat, axis=0, keepdims=True)
    acc_d[0] = acc_d[0] + jnp.sum(diag)

    @pl.when(i == nblk - 1)
    def _():
        total = (jnp.float32(_B) * jnp.float32(_B)
                 + jnp.sum(acc_rep[...] * acc_rec[...])
                 - 2.0 * acc_d[0])
        out_ref[...] = jnp.full((1, 1), total, jnp.float32)


def _tc_loss(tsum, bsum, title_int, body_int, C):
    blk = 512
    nblk = _B // blk
    return pl.pallas_call(
        functools.partial(_tc_body, nblk),
        grid=(nblk,),
        in_specs=[
            pl.BlockSpec((blk, _DIM), lambda i: (i, 0)),
            pl.BlockSpec((blk, _DIM), lambda i: (i, 0)),
            pl.BlockSpec((blk, _T_LEN), lambda i: (i, 0)),
            pl.BlockSpec((blk, _BODY_LEN), lambda i: (i, 0)),
            pl.BlockSpec((_NUM_C, _DIM), lambda i: (0, 0)),
        ],
        out_specs=pl.BlockSpec((1, 1), lambda i: (0, 0)),
        out_shape=jax.ShapeDtypeStruct((1, 1), jnp.float32),
        scratch_shapes=[
            pltpu.VMEM((1, _DIM), jnp.float32),
            pltpu.VMEM((1, _DIM), jnp.float32),
            pltpu.SMEM((1,), jnp.float32),
        ],
        compiler_params=pltpu.CompilerParams(
            dimension_semantics=("arbitrary",)),
    )(tsum, bsum, title_int, body_int, C)


def kernel(title_int, body_int, user_int, W, C):
    title2 = title_int.reshape(_B * _T_LEN).astype(jnp.int32)
    body2 = body_int.reshape(_B * _BODY_LEN).astype(jnp.int32)
    scale = jnp.float32(1.0) + jnp.float32(0.0) * user_int[0].astype(
        jnp.float32)
    tsum, bsum = _sc_pool(W * scale, title2, body2)
    out = _tc_loss(tsum, bsum, title_int.astype(jnp.int32),
                   body_int.astype(jnp.int32), C)
    return out[0, 0]


# speedup vs baseline: 1.9701x; 1.3915x over previous
import functools

import jax
import jax.numpy as jnp
from jax import lax
from jax.experimental import pallas as pl
from jax.experimental.pallas import tpu as pltpu
from jax.experimental.pallas import tpu_sc as plsc

_NUM_W = 1000000
_DIM = 64
_NUM_C = 1024
_B = 4096
_T_LEN = 20
_BODY_LEN = 200

_NC, _NS = 2, 16
_NW = _NC * _NS
_SPW = _B // _NW
_PAIRS = _SPW // 2
_TPP = 2 * _T_LEN
_BPP = 2 * _BODY_LEN
_BCH = 80
_NBCH = _BPP // _BCH


def _sc_body(w_hbm, tidx_hbm, bidx_hbm, tsum_hbm, bsum_hbm,
             tidx_v, bidx_v, trows_v, brows_v, tout_v, bout_v, sem):
    wid = lax.axis_index("s") * _NC + lax.axis_index("c")
    sbase = wid * _SPW
    pltpu.sync_copy(tidx_hbm.at[pl.ds(wid * _SPW * _T_LEN, _SPW * _T_LEN)],
                    tidx_v)
    pltpu.sync_copy(bidx_hbm.at[pl.ds(wid * _SPW * _BODY_LEN,
                                      _SPW * _BODY_LEN)], bidx_v)

    def pair_body(j, carry):
        toff = pl.multiple_of(j * _TPP, 8)
        cps = [pltpu.async_copy(w_hbm.at[tidx_v.at[pl.ds(toff, _TPP)]],
                                trows_v, sem)]
        for c in range(_NBCH):
            boff = pl.multiple_of(j * _BPP + c * _BCH, 8)
            cps.append(pltpu.async_copy(
                w_hbm.at[bidx_v.at[pl.ds(boff, _BCH)]],
                brows_v.at[pl.ds(c * _BCH, _BCH)], sem))
        for cp in cps:
            cp.wait()
        for s in range(2):
            tacc = [jnp.zeros((16,), jnp.float32) for _ in range(4)]
            for r in range(_T_LEN):
                for d in range(4):
                    tacc[d] = tacc[d] + trows_v[s * _T_LEN + r,
                                                pl.ds(d * 16, 16)]
            def red8(it, accs):
                a = list(accs)
                for k in range(8):
                    r = s * _BODY_LEN + it * 8 + k
                    for d in range(4):
                        a[d] = a[d] + brows_v[r, pl.ds(d * 16, 16)]
                return tuple(a)
            bacc = lax.fori_loop(
                0, _BODY_LEN // 8, red8,
                tuple(jnp.zeros((16,), jnp.float32) for _ in range(4)))
            row = 2 * j + s
            for d in range(4):
                tout_v[row, pl.ds(d * 16, 16)] = tacc[d]
                bout_v[row, pl.ds(d * 16, 16)] = bacc[d]
        return carry

    lax.fori_loop(0, _PAIRS, pair_body, 0)
    pltpu.sync_copy(tout_v, tsum_hbm.at[pl.ds(sbase, _SPW)])
    pltpu.sync_copy(bout_v, bsum_hbm.at[pl.ds(sbase, _SPW)])


def _sc_pool(W, title2, body2):
    mesh = plsc.VectorSubcoreMesh(core_axis_name="c", subcore_axis_name="s",
                                  num_cores=_NC, num_subcores=_NS)
    f = pl.kernel(
        _sc_body,
        out_type=(jax.ShapeDtypeStruct((_B, _DIM), jnp.float32),
                  jax.ShapeDtypeStruct((_B, _DIM), jnp.float32)),
        mesh=mesh,
        scratch_types=[
            pltpu.VMEM((_SPW * _T_LEN,), jnp.int32),
            pltpu.VMEM((_SPW * _BODY_LEN,), jnp.int32),
            pltpu.VMEM((_TPP, _DIM), jnp.float32),
            pltpu.VMEM((_BPP, _DIM), jnp.float32),
            pltpu.VMEM((_SPW, _DIM), jnp.float32),
            pltpu.VMEM((_SPW, _DIM), jnp.float32),
            pltpu.SemaphoreType.DMA,
        ],
        compiler_params=pltpu.CompilerParams(use_tc_tiling_on_sc=False),
    )
    return f(W, title2, body2)


def _tc_body(nblk, tsum_ref, bsum_ref, tint_ref, bint_ref, c_ref, out_ref,
             acc_rep, acc_rec, acc_d):
    i = pl.program_id(0)

    @pl.when(i == 0)
    def _():
        acc_rep[...] = jnp.zeros_like(acc_rep)
        acc_rec[...] = jnp.zeros_like(acc_rec)
        acc_d[0] = 0.0

    tcnt = jnp.sum((tint_ref[...] > 0).astype(jnp.float32), axis=1,
                   keepdims=True)
    bcnt = jnp.sum((bint_ref[...] > 0).astype(jnp.float32), axis=1,
                   keepdims=True)
    que = 0.3 * tsum_ref[...] / tcnt + 0.7 * bsum_ref[...] / bcnt
    cmat = c_ref[...]
    score = lax.dot_general(que, cmat, (((1,), (1,)), ((), ())),
                            preferred_element_type=jnp.float32)
    m = jnp.max(score, axis=1, keepdims=True)
    e = jnp.exp(score - m)
    probs = e / jnp.sum(e, axis=1, keepdims=True)
    rec = lax.dot_general(probs, cmat, (((1,), (0,)), ((), ())),
                          preferred_element_type=jnp.float32)
    n_rep = jnp.sqrt(jnp.sum(que * que, axis=1, keepdims=True))
    n_rec = jnp.sqrt(jnp.sum(rec * rec, axis=1, keepdims=True))
    denom = jnp.maximum(n_rec * n_rep, 1e-8)
    diag = jnp.sum(rec * que, axis=1, keepdims=True) / denom
    rep_hat = que / jnp.maximum(n_rep, 1e-20)
    rec_hat = rec / jnp.maximum(n_rec, 1e-20)

    acc_rep[...] = acc_rep[...] + jnp.sum(rep_hat, axis=0, keepdims=True)
    acc_rec[...] = acc_rec[...] + jnp.sum(rec_hat, axis=0, keepdims=True)
    acc_d[0] = acc_d[0] + jnp.sum(diag)

    @pl.when(i == nblk - 1)
    def _():
        total = (jnp.float32(_B) * jnp.float32(_B)
                 + jnp.sum(acc_rep[...] * acc_rec[...])
                 - 2.0 * acc_d[0])
        out_ref[...] = jnp.full((1, 1), total, jnp.float32)


def _tc_loss(tsum, bsum, title_int, body_int, C):
    blk = 512
    nblk = _B // blk
    return pl.pallas_call(
        functools.partial(_tc_body, nblk),
        grid=(nblk,),
        in_specs=[
            pl.BlockSpec((blk, _DIM), lambda i: (i, 0)),
            pl.BlockSpec((blk, _DIM), lambda i: (i, 0)),
            pl.BlockSpec((blk, _T_LEN), lambda i: (i, 0)),
            pl.BlockSpec((blk, _BODY_LEN), lambda i: (i, 0)),
            pl.BlockSpec((_NUM_C, _DIM), lambda i: (0, 0)),
        ],
        out_specs=pl.BlockSpec((1, 1), lambda i: (0, 0)),
        out_shape=jax.ShapeDtypeStruct((1, 1), jnp.float32),
        scratch_shapes=[
            pltpu.VMEM((1, _DIM), jnp.float32),
            pltpu.VMEM((1, _DIM), jnp.float32),
            pltpu.SMEM((1,), jnp.float32),
        ],
        compiler_params=pltpu.CompilerParams(
            dimension_semantics=("arbitrary",)),
    )(tsum, bsum, title_int, body_int, C)


def kernel(title_int, body_int, user_int, W, C):
    title2 = title_int.reshape(_B * _T_LEN).astype(jnp.int32)
    body2 = body_int.reshape(_B * _BODY_LEN).astype(jnp.int32)
    tsum, bsum = _sc_pool(W, title2, body2)
    out = _tc_loss(tsum, bsum, title_int.astype(jnp.int32),
                   body_int.astype(jnp.int32), C)
    return out[0, 0]
